# baseline (speedup 1.0000x reference)
import functools

import jax
import jax.numpy as jnp
from jax import lax
from jax.experimental import pallas as pl
from jax.experimental.pallas import tpu as pltpu


def _fused_body(media_ref, WbigT_ref, bbig_ref, WwT_ref, bw_ref,
                W2T_ref, b2_ref, W3T_ref, b3_ref, start_ref, end_ref,
                out_ref, zo_ref, *, blk, nsteps, nseg, ncls, h, d1):
    i = pl.program_id(0)

    @pl.when(i == 0)
    def _init():
        zo_ref[...] = jnp.zeros((2 * ncls, nseg), jnp.float32)

    x = media_ref[...]
    ag = jnp.dot(x, WbigT_ref[...], preferred_element_type=jnp.float32) \
        + bbig_ref[...]
    a = jnp.tanh(ag[:, :h])
    g = 0.5 * (1.0 + jnp.tanh(ag[:, h:2 * h] * 0.5))
    h1 = jax.nn.gelu(ag[:, 2 * h:])
    s = jnp.dot(a * g, WwT_ref[...], preferred_element_type=jnp.float32) \
        + bw_ref[...]
    h2 = jax.nn.gelu(jnp.dot(h1, W2T_ref[...],
                             preferred_element_type=jnp.float32) + b2_ref[...])
    logit = jnp.dot(h2, W3T_ref[...], preferred_element_type=jnp.float32) \
        + b3_ref[...]

    tok = i * blk + lax.broadcasted_iota(jnp.int32, (blk, nseg), 0)
    onehot = ((tok >= start_ref[...]) & (tok < end_ref[...])) \
        .astype(jnp.float32)

    e = jnp.exp(s)
    q = jnp.concatenate([e, e * logit], axis=1)
    zo_ref[...] += lax.dot_general(q, onehot, (((0,), (0,)), ((), ())),
                                   preferred_element_type=jnp.float32)

    @pl.when(i == nsteps - 1)
    def _fin():
        z = zo_ref[:ncls, :]
        o = zo_ref[ncls:, :]
        out_ref[...] = jnp.where(z > 0, o / z, 0.0)


def kernel(media, cu_seqlens, Wa, ba, Wg, bg, Ww, bw, W1, b1, W2, b2, W3, b3,
           output_scale, output_bias):
    n_tok, d = media.shape
    nseg = cu_seqlens.shape[0] - 1
    ncls = Ww.shape[0]
    h = Wa.shape[0]
    d1 = W1.shape[0]
    d2 = W2.shape[0]
    blk = 2048
    nsteps = n_tok // blk
    dbig = 2 * h + d1

    body = functools.partial(_fused_body, blk=blk, nsteps=nsteps, nseg=nseg,
                             ncls=ncls, h=h, d1=d1)
    row = lambda v: v.reshape(1, -1)
    WbigT = jnp.concatenate([Wa.T, Wg.T, W1.T], axis=1)
    bbig = jnp.concatenate([ba, bg, b1])
    start = cu_seqlens[:nseg].reshape(1, nseg)
    end = cu_seqlens[1:].reshape(1, nseg)
    const = lambda shape: pl.BlockSpec(shape, lambda i: (0, 0))
    out = pl.pallas_call(
        body,
        grid=(nsteps,),
        in_specs=[
            pl.BlockSpec((blk, d), lambda i: (i, 0)),
            const((d, dbig)), const((1, dbig)),
            const((d, ncls)), const((1, ncls)),
            const((d1, d2)), const((1, d2)),
            const((d2, ncls)), const((1, ncls)),
            const((1, nseg)), const((1, nseg)),
        ],
        out_specs=pl.BlockSpec((ncls, nseg), lambda i: (0, 0)),
        out_shape=jax.ShapeDtypeStruct((ncls, nseg), jnp.float32),
        scratch_shapes=[pltpu.VMEM((2 * ncls, nseg), jnp.float32)],
    )(media, WbigT, row(bbig), Ww.T, row(bw),
      W2.T, row(b2), W3.T, row(b3), start, end)
    return out.T * output_scale + output_bias


# speedup vs baseline: 1.0212x; 1.0212x over previous
import functools

import jax
import jax.numpy as jnp
from jax import lax
from jax.experimental import pallas as pl
from jax.experimental.pallas import tpu as pltpu


def _fused_body(media_ref, WbigT_ref, bbig_ref, WwT_ref, bw_ref,
                W2T_ref, b2_ref, W3T_ref, b3_ref, start_ref, end_ref,
                out_ref, zo_ref, *, blk, nsteps, nseg, ncls, h, d1):
    i = pl.program_id(0)

    @pl.when(i == 0)
    def _init():
        zo_ref[...] = jnp.zeros((2 * ncls, nseg), jnp.float32)

    x = media_ref[...]
    ag = jnp.dot(x, WbigT_ref[...], preferred_element_type=jnp.float32) \
        + bbig_ref[...]
    a = jnp.tanh(ag[:, :h])
    g = 0.5 * (1.0 + jnp.tanh(ag[:, h:2 * h] * 0.5))
    h1 = jax.nn.gelu(ag[:, 2 * h:])
    s = jnp.dot(a * g, WwT_ref[...], preferred_element_type=jnp.float32) \
        + bw_ref[...]
    h2 = jax.nn.gelu(jnp.dot(h1, W2T_ref[...],
                             preferred_element_type=jnp.float32) + b2_ref[...])
    logit = jnp.dot(h2, W3T_ref[...], preferred_element_type=jnp.float32) \
        + b3_ref[...]

    tok = i * blk + lax.broadcasted_iota(jnp.int32, (blk, nseg), 0)
    onehot = ((tok >= start_ref[...]) & (tok < end_ref[...])) \
        .astype(jnp.float32)

    e = jnp.exp(s)
    q = jnp.concatenate([e, e * logit], axis=1)
    zo_ref[...] += lax.dot_general(q, onehot, (((0,), (0,)), ((), ())),
                                   preferred_element_type=jnp.float32)

    @pl.when(i == nsteps - 1)
    def _fin():
        z = zo_ref[:ncls, :]
        o = zo_ref[ncls:, :]
        out_ref[...] = jnp.where(z > 0, o / z, 0.0)


def kernel(media, cu_seqlens, Wa, ba, Wg, bg, Ww, bw, W1, b1, W2, b2, W3, b3,
           output_scale, output_bias):
    n_tok, d = media.shape
    nseg = cu_seqlens.shape[0] - 1
    ncls = Ww.shape[0]
    h = Wa.shape[0]
    d1 = W1.shape[0]
    d2 = W2.shape[0]
    blk = 1024
    nsteps = n_tok // blk
    dbig = 2 * h + d1

    body = functools.partial(_fused_body, blk=blk, nsteps=nsteps, nseg=nseg,
                             ncls=ncls, h=h, d1=d1)
    row = lambda v: v.reshape(1, -1)
    WbigT = jnp.concatenate([Wa.T, Wg.T, W1.T], axis=1)
    bbig = jnp.concatenate([ba, bg, b1])
    start = cu_seqlens[:nseg].reshape(1, nseg)
    end = cu_seqlens[1:].reshape(1, nseg)
    const = lambda shape: pl.BlockSpec(shape, lambda i: (0, 0))
    out = pl.pallas_call(
        body,
        grid=(nsteps,),
        in_specs=[
            pl.BlockSpec((blk, d), lambda i: (i, 0)),
            const((d, dbig)), const((1, dbig)),
            const((d, ncls)), const((1, ncls)),
            const((d1, d2)), const((1, d2)),
            const((d2, ncls)), const((1, ncls)),
            const((1, nseg)), const((1, nseg)),
        ],
        out_specs=pl.BlockSpec((ncls, nseg), lambda i: (0, 0)),
        out_shape=jax.ShapeDtypeStruct((ncls, nseg), jnp.float32),
        scratch_shapes=[pltpu.VMEM((2 * ncls, nseg), jnp.float32)],
        compiler_params=pltpu.CompilerParams(fuse_transposed_lhs_in_matmul=True),
    )(media, WbigT, row(bbig), Ww.T, row(bw),
      W2.T, row(b2), W3.T, row(b3), start, end)
    return out.T * output_scale + output_bias


# speedup vs baseline: 1.0226x; 1.0014x over previous
import functools

import jax
import jax.numpy as jnp
from jax import lax
from jax.experimental import pallas as pl
from jax.experimental.pallas import tpu as pltpu


def _fused_body(media_ref, WbigT_ref, bbig_ref, WwT_ref, bw_ref,
                W2T_ref, b2_ref, W3T_ref, b3_ref, start_ref, end_ref,
                out_ref, zo_ref, *, blk, nsteps, nseg, ncls, h, d1):
    i = pl.program_id(0)

    @pl.when(i == 0)
    def _init():
        zo_ref[...] = jnp.zeros((2 * ncls, nseg), jnp.float32)

    x = media_ref[...]
    ag = jnp.dot(x, WbigT_ref[...], preferred_element_type=jnp.float32) \
        + bbig_ref[...]
    a = jnp.tanh(ag[:, :h])
    g = 0.5 * (1.0 + jnp.tanh(ag[:, h:2 * h] * 0.5))
    h1 = jax.nn.gelu(ag[:, 2 * h:])
    s = jnp.dot(a * g, WwT_ref[...], preferred_element_type=jnp.float32) \
        + bw_ref[...]
    h2 = jax.nn.gelu(jnp.dot(h1, W2T_ref[...],
                             preferred_element_type=jnp.float32) + b2_ref[...])
    logit = jnp.dot(h2, W3T_ref[...], preferred_element_type=jnp.float32) \
        + b3_ref[...]

    tok = i * blk + lax.broadcasted_iota(jnp.int32, (blk, nseg), 0)
    onehot = ((tok >= start_ref[...]) & (tok < end_ref[...])) \
        .astype(jnp.float32)

    e = jnp.exp(s)
    q = jnp.concatenate([e, e * logit], axis=1)
    zo_ref[...] += lax.dot_general(q, onehot, (((0,), (0,)), ((), ())),
                                   preferred_element_type=jnp.float32)

    @pl.when(i == nsteps - 1)
    def _fin():
        z = zo_ref[:ncls, :]
        o = zo_ref[ncls:, :]
        out_ref[...] = jnp.where(z > 0, o / z, 0.0)


def kernel(media, cu_seqlens, Wa, ba, Wg, bg, Ww, bw, W1, b1, W2, b2, W3, b3,
           output_scale, output_bias):
    n_tok, d = media.shape
    nseg = cu_seqlens.shape[0] - 1
    ncls = Ww.shape[0]
    h = Wa.shape[0]
    d1 = W1.shape[0]
    d2 = W2.shape[0]
    blk = 1024
    nsteps = n_tok // blk
    dbig = 2 * h + d1

    body = functools.partial(_fused_body, blk=blk, nsteps=nsteps, nseg=nseg,
                             ncls=ncls, h=h, d1=d1)
    row = lambda v: v.reshape(1, -1)
    WbigT = jnp.concatenate([Wa.T, Wg.T, W1.T], axis=1)
    bbig = jnp.concatenate([ba, bg, b1])
    start = cu_seqlens[:nseg].reshape(1, nseg)
    end = cu_seqlens[1:].reshape(1, nseg)
    const = lambda shape: pl.BlockSpec(shape, lambda i: (0, 0))
    out = pl.pallas_call(
        body,
        grid=(nsteps,),
        in_specs=[
            pl.BlockSpec((blk, d), lambda i: (i, 0)),
            const((d, dbig)), const((1, dbig)),
            const((d, ncls)), const((1, ncls)),
            const((d1, d2)), const((1, d2)),
            const((d2, ncls)), const((1, ncls)),
            const((1, nseg)), const((1, nseg)),
        ],
        out_specs=pl.BlockSpec((ncls, nseg), lambda i: (0, 0)),
        out_shape=jax.ShapeDtypeStruct((ncls, nseg), jnp.float32),
        scratch_shapes=[pltpu.VMEM((2 * ncls, nseg), jnp.float32)],
    )(media, WbigT, row(bbig), Ww.T, row(bw),
      W2.T, row(b2), W3.T, row(b3), start, end)
    return out.T * output_scale + output_bias


# speedup vs baseline: 1.0593x; 1.0358x over previous
import functools

import jax
import jax.numpy as jnp
from jax import lax
from jax.experimental import pallas as pl
from jax.experimental.pallas import tpu as pltpu


def _fused_body(media_ref, Wbig_ref, bbig_ref, Ww_ref, bw_ref,
                W2_ref, b2_ref, W3_ref, b3_ref, start_ref, end_ref,
                out_ref, zo_ref, *, blk, nsteps, nseg, ncls, h, d1):
    i = pl.program_id(0)

    @pl.when(i == 0)
    def _init():
        zo_ref[...] = jnp.zeros((2 * ncls, nseg), jnp.float32)

    x = media_ref[...]
    nt = (((1,), (1,)), ((), ()))
    ag = lax.dot_general(x, Wbig_ref[...], nt,
                         preferred_element_type=jnp.float32) + bbig_ref[...]
    a = jnp.tanh(ag[:, :h])
    g = 0.5 * (1.0 + jnp.tanh(ag[:, h:2 * h] * 0.5))
    h1 = jax.nn.gelu(ag[:, 2 * h:])
    s = lax.dot_general(a * g, Ww_ref[...], nt,
                        preferred_element_type=jnp.float32) + bw_ref[...]
    h2 = jax.nn.gelu(lax.dot_general(h1, W2_ref[...], nt,
                     preferred_element_type=jnp.float32) + b2_ref[...])
    logit = lax.dot_general(h2, W3_ref[...], nt,
                            preferred_element_type=jnp.float32) + b3_ref[...]

    tok = i * blk + lax.broadcasted_iota(jnp.int32, (blk, nseg), 0)
    onehot = ((tok >= start_ref[...]) & (tok < end_ref[...])) \
        .astype(jnp.float32)

    e = jnp.exp(s)
    q = jnp.concatenate([e, e * logit], axis=1)
    zo_ref[...] += lax.dot_general(q, onehot, (((0,), (0,)), ((), ())),
                                   preferred_element_type=jnp.float32)

    @pl.when(i == nsteps - 1)
    def _fin():
        z = zo_ref[:ncls, :]
        o = zo_ref[ncls:, :]
        out_ref[...] = jnp.where(z > 0, o / z, 0.0)


def kernel(media, cu_seqlens, Wa, ba, Wg, bg, Ww, bw, W1, b1, W2, b2, W3, b3,
           output_scale, output_bias):
    n_tok, d = media.shape
    nseg = cu_seqlens.shape[0] - 1
    ncls = Ww.shape[0]
    h = Wa.shape[0]
    d1 = W1.shape[0]
    d2 = W2.shape[0]
    blk = 1024
    nsteps = n_tok // blk
    dbig = 2 * h + d1

    body = functools.partial(_fused_body, blk=blk, nsteps=nsteps, nseg=nseg,
                             ncls=ncls, h=h, d1=d1)
    row = lambda v: v.reshape(1, -1)
    Wbig = jnp.concatenate([Wa, Wg, W1], axis=0)
    bbig = jnp.concatenate([ba, bg, b1])
    start = cu_seqlens[:nseg].reshape(1, nseg)
    end = cu_seqlens[1:].reshape(1, nseg)
    const = lambda shape: pl.BlockSpec(shape, lambda i: (0, 0))
    out = pl.pallas_call(
        body,
        grid=(nsteps,),
        in_specs=[
            pl.BlockSpec((blk, d), lambda i: (i, 0)),
            const((dbig, d)), const((1, dbig)),
            const((ncls, d)), const((1, ncls)),
            const((d2, d1)), const((1, d2)),
            const((ncls, d2)), const((1, ncls)),
            const((1, nseg)), const((1, nseg)),
        ],
        out_specs=pl.BlockSpec((ncls, nseg), lambda i: (0, 0)),
        out_shape=jax.ShapeDtypeStruct((ncls, nseg), jnp.float32),
        scratch_shapes=[pltpu.VMEM((2 * ncls, nseg), jnp.float32)],
    )(media, Wbig, row(bbig), Ww, row(bw),
      W2, row(b2), W3, row(b3), start, end)
    return out.T * output_scale + output_bias


# speedup vs baseline: 1.0908x; 1.0297x over previous
import functools

import jax
import jax.numpy as jnp
from jax import lax
from jax.experimental import pallas as pl
from jax.experimental.pallas import tpu as pltpu


def _fused_body(media_ref, Wa_ref, ba_ref, Wg_ref, bg_ref, W1_ref,
                b1_ref, Ww_ref, bw_ref,
                W2_ref, b2_ref, W3_ref, b3_ref, start_ref, end_ref,
                out_ref, zo_ref, *, blk, nsteps, nseg, ncls, h, d1):
    i = pl.program_id(0)

    @pl.when(i == 0)
    def _init():
        zo_ref[...] = jnp.zeros((2 * ncls, nseg), jnp.float32)

    x = media_ref[...]
    nt = (((1,), (1,)), ((), ()))
    a = jnp.tanh(lax.dot_general(x, Wa_ref[...], nt,
                                 preferred_element_type=jnp.float32)
                 + ba_ref[...])
    g = 0.5 * (1.0 + jnp.tanh(
        (lax.dot_general(x, Wg_ref[...], nt,
                         preferred_element_type=jnp.float32)
         + bg_ref[...]) * 0.5))
    h1 = jax.nn.gelu(lax.dot_general(x, W1_ref[...], nt,
                                     preferred_element_type=jnp.float32)
                     + b1_ref[...])
    s = lax.dot_general(a * g, Ww_ref[...], nt,
                        preferred_element_type=jnp.float32) + bw_ref[...]
    h2 = jax.nn.gelu(lax.dot_general(h1, W2_ref[...], nt,
                     preferred_element_type=jnp.float32) + b2_ref[...])
    logit = lax.dot_general(h2, W3_ref[...], nt,
                            preferred_element_type=jnp.float32) + b3_ref[...]

    tok = i * blk + lax.broadcasted_iota(jnp.int32, (blk, nseg), 0)
    onehot = ((tok >= start_ref[...]) & (tok < end_ref[...])) \
        .astype(jnp.float32)

    e = jnp.exp(s)
    q = jnp.concatenate([e, e * logit], axis=1)
    zo_ref[...] += lax.dot_general(q, onehot, (((0,), (0,)), ((), ())),
                                   preferred_element_type=jnp.float32)

    @pl.when(i == nsteps - 1)
    def _fin():
        z = zo_ref[:ncls, :]
        o = zo_ref[ncls:, :]
        out_ref[...] = jnp.where(z > 0, o / z, 0.0)


def kernel(media, cu_seqlens, Wa, ba, Wg, bg, Ww, bw, W1, b1, W2, b2, W3, b3,
           output_scale, output_bias):
    n_tok, d = media.shape
    nseg = cu_seqlens.shape[0] - 1
    ncls = Ww.shape[0]
    h = Wa.shape[0]
    d1 = W1.shape[0]
    d2 = W2.shape[0]
    blk = 1024
    nsteps = n_tok // blk
    dbig = 2 * h + d1

    body = functools.partial(_fused_body, blk=blk, nsteps=nsteps, nseg=nseg,
                             ncls=ncls, h=h, d1=d1)
    row = lambda v: v.reshape(1, -1)
    start = cu_seqlens[:nseg].reshape(1, nseg)
    end = cu_seqlens[1:].reshape(1, nseg)
    const = lambda shape: pl.BlockSpec(shape, lambda i: (0, 0))
    out = pl.pallas_call(
        body,
        grid=(nsteps,),
        in_specs=[
            pl.BlockSpec((blk, d), lambda i: (i, 0)),
            const((h, d)), const((1, h)),
            const((h, d)), const((1, h)),
            const((d1, d)), const((1, d1)),
            const((ncls, d)), const((1, ncls)),
            const((d2, d1)), const((1, d2)),
            const((ncls, d2)), const((1, ncls)),
            const((1, nseg)), const((1, nseg)),
        ],
        out_specs=pl.BlockSpec((ncls, nseg), lambda i: (0, 0)),
        out_shape=jax.ShapeDtypeStruct((ncls, nseg), jnp.float32),
        scratch_shapes=[pltpu.VMEM((2 * ncls, nseg), jnp.float32)],
    )(media, Wa, row(ba), Wg, row(bg), W1, row(b1), Ww, row(bw),
      W2, row(b2), W3, row(b3), start, end)
    return out.T * output_scale + output_bias
